# Initial kernel scaffold; baseline (speedup 1.0000x reference)
import jax
import jax.numpy as jnp
from jax.experimental import pallas as pl


def kernel(token_ids, table):
    raise NotImplementedError("write your pallas kernel here")


# speedup vs baseline: 3.9530x; 3.9530x over previous
import functools

import jax
import jax.numpy as jnp
from jax import lax
from jax.experimental import pallas as pl
from jax.experimental.pallas import tpu as pltpu
from jax.experimental.pallas import tpu_sc as plsc

EMB = 64
NUM_CORES = 2
NUM_SUBCORES = 16
NW = NUM_CORES * NUM_SUBCORES
CHUNK = 512


def _make_gather(total: int):
    per_w = total // NW
    n_chunk = per_w // CHUNK
    mesh = plsc.VectorSubcoreMesh(
        core_axis_name="c", subcore_axis_name="s",
        num_cores=NUM_CORES, num_subcores=NUM_SUBCORES)

    @functools.partial(
        pl.kernel,
        out_type=jax.ShapeDtypeStruct((total, EMB), jnp.float32),
        mesh=mesh,
        scratch_types=[
            pltpu.VMEM((CHUNK,), jnp.int32),
            pltpu.VMEM((CHUNK, EMB), jnp.float32),
            pltpu.SemaphoreType.DMA,
        ],
        compiler_params=pltpu.CompilerParams(use_tc_tiling_on_sc=False),
    )
    def gather(ids_hbm, table_hbm, out_hbm, idx_v, rows_v, sem):
        wid = lax.axis_index("s") * NUM_CORES + lax.axis_index("c")
        base = wid * per_w

        def body(i, carry):
            off = base + i * CHUNK
            pltpu.sync_copy(ids_hbm.at[pl.ds(off, CHUNK)], idx_v)
            pltpu.async_copy(table_hbm.at[idx_v], rows_v, sem).wait()
            pltpu.sync_copy(rows_v, out_hbm.at[pl.ds(off, CHUNK)])
            return carry

        lax.fori_loop(0, n_chunk, body, 0)

    return gather


def kernel(token_ids, table):
    b, l = token_ids.shape
    flat = token_ids.reshape(-1)
    out = _make_gather(b * l)(flat, table)
    return out.reshape(b, l, EMB)


# speedup vs baseline: 4.2509x; 1.0754x over previous
import functools

import jax
import jax.numpy as jnp
from jax import lax
from jax.experimental import pallas as pl
from jax.experimental.pallas import tpu as pltpu
from jax.experimental.pallas import tpu_sc as plsc

EMB = 64
NUM_CORES = 2
NUM_SUBCORES = 16
NW = NUM_CORES * NUM_SUBCORES
CHUNK = 800
NBUF = 2


def _make_gather(total: int):
    per_w = total // NW
    n_chunk = per_w // CHUNK
    n_outer = n_chunk // NBUF
    mesh = plsc.VectorSubcoreMesh(
        core_axis_name="c", subcore_axis_name="s",
        num_cores=NUM_CORES, num_subcores=NUM_SUBCORES)

    @functools.partial(
        pl.kernel,
        out_type=jax.ShapeDtypeStruct((total, EMB), jnp.float32),
        mesh=mesh,
        scratch_types=[
            [pltpu.VMEM((CHUNK,), jnp.int32) for _ in range(NBUF)],
            [pltpu.VMEM((CHUNK, EMB), jnp.float32) for _ in range(NBUF)],
            pltpu.SemaphoreType.DMA,
            pltpu.SemaphoreType.DMA,
            pltpu.SemaphoreType.DMA,
        ],
        compiler_params=pltpu.CompilerParams(use_tc_tiling_on_sc=False),
    )
    def gather(ids_hbm, table_hbm, out_hbm, idx_v, rows_v, idx_sem, gat_sem,
               out_sem):
        wid = lax.axis_index("s") * NUM_CORES + lax.axis_index("c")
        base = wid * per_w

        for b in range(NBUF):
            pltpu.async_copy(
                ids_hbm.at[pl.ds(base + b * CHUNK, CHUNK)], idx_v[b], idx_sem)

        def body(g, carry):
            for b in range(NBUF):
                i = g * NBUF + b
                off = base + i * CHUNK
                pltpu.make_async_copy(
                    ids_hbm.at[pl.ds(base, CHUNK)], idx_v[b], idx_sem).wait()
                @pl.when(g >= 1)
                def _():
                    pltpu.make_async_copy(
                        rows_v[b], out_hbm.at[pl.ds(base, CHUNK)],
                        out_sem).wait()
                pltpu.async_copy(
                    table_hbm.at[idx_v[b]], rows_v[b], gat_sem).wait()
                pltpu.async_copy(
                    rows_v[b], out_hbm.at[pl.ds(off, CHUNK)], out_sem)
                @pl.when(g < n_outer - 1)
                def _():
                    pltpu.async_copy(
                        ids_hbm.at[pl.ds(off + NBUF * CHUNK, CHUNK)],
                        idx_v[b], idx_sem)
            return carry

        lax.fori_loop(0, n_outer, body, 0)

        for b in range(NBUF):
            pltpu.make_async_copy(
                rows_v[b], out_hbm.at[pl.ds(base, CHUNK)], out_sem).wait()

    return gather


def kernel(token_ids, table):
    b, l = token_ids.shape
    flat = token_ids.reshape(-1)
    out = _make_gather(b * l)(flat, table)
    return out.reshape(b, l, EMB)
